# baseline (speedup 1.0000x reference)
import functools

import jax
import jax.numpy as jnp
from jax import lax
from jax.experimental import pallas as pl
from jax.experimental.pallas import tpu as pltpu
from jax.experimental.pallas import tpu_sc as plsc

_NC = 2
_NS = 16
_L = 16

_CHUNK = 2000
_NBUF = 2


def _sc_gather_scatter(featP_flat, edge_flat, n_nodes, d_in):
    cols_per_w = d_in // _NS
    pairs_per_w = cols_per_w // 2
    n_edges = edge_flat.shape[0] // 2
    e_half = n_edges // _NC
    n_chunks = e_half // _CHUNK
    mesh = plsc.VectorSubcoreMesh(core_axis_name="c", subcore_axis_name="s")

    out_sds = jax.ShapeDtypeStruct((d_in * n_nodes,), jnp.float32)
    scratch = (
        [pltpu.VMEM((n_nodes,), jnp.int32) for _ in range(pairs_per_w)]
        + [pltpu.VMEM((n_nodes,), jnp.float32) for _ in range(cols_per_w)]
        + [pltpu.VMEM((_CHUNK,), jnp.int32) for _ in range(2 * _NBUF)]
        + [pltpu.SemaphoreType.DMA, pltpu.SemaphoreType.DMA,
           pltpu.SemaphoreType.DMA]
    )

    @functools.partial(
        pl.kernel,
        out_type=(out_sds, out_sds),
        mesh=mesh,
        scratch_types=scratch,
        compiler_params=pltpu.CompilerParams(needs_layout_passes=False),
    )
    def k(featP_hbm, edge_hbm, agg0_hbm, agg1_hbm,
          f0, f1, f2, f3, a0, a1, a2, a3, a4, a5, a6, a7,
          s0, s1, t0, t1, sem0, sem1, semf):
        feat_pairs = (f0, f1, f2, f3)
        agg_cols = (a0, a1, a2, a3, a4, a5, a6, a7)
        src_bufs = (s0, s1)
        dst_bufs = (t0, t1)
        sems = (sem0, sem1)
        cid = lax.axis_index("c")
        sid = lax.axis_index("s")
        ebase = cid * e_half
        prow0 = sid * pairs_per_w
        row0 = sid * cols_per_w

        def start(b, ck):
            off = ebase + ck * _CHUNK
            pltpu.async_copy(edge_hbm.at[pl.ds(off, _CHUNK)], src_bufs[b], sems[b])
            pltpu.async_copy(edge_hbm.at[pl.ds(n_edges + off, _CHUNK)],
                             dst_bufs[b], sems[b])

        def drain(b):
            pltpu.make_async_copy(edge_hbm.at[pl.ds(0, _CHUNK)], src_bufs[b], sems[b]).wait()
            pltpu.make_async_copy(edge_hbm.at[pl.ds(0, _CHUNK)], dst_bufs[b], sems[b]).wait()

        for b in range(_NBUF):
            start(b, b)
        stage = [
            pltpu.async_copy(
                featP_hbm.at[pl.ds((prow0 + p) * n_nodes, n_nodes)],
                feat_pairs[p], semf)
            for p in range(pairs_per_w)
        ]

        @plsc.parallel_loop(0, n_nodes // _L, unroll=8)
        def _zero(i):
            for c in range(cols_per_w):
                agg_cols[c][pl.ds(i * _L, _L)] = jnp.zeros((_L,), jnp.float32)

        for d in stage:
            d.wait()

        himask = jnp.full((_L,), jnp.int32(-65536))

        @pl.loop(0, n_chunks // _NBUF)
        def _outer(g):
            for b in range(_NBUF):
                ck = g * _NBUF + b
                drain(b)

                @plsc.parallel_loop(0, _CHUNK // _L, unroll=16)
                def _edges(i):
                    s = src_bufs[b][pl.ds(i * _L, _L)]
                    t = dst_bufs[b][pl.ds(i * _L, _L)]
                    for p in range(pairs_per_w):
                        g32 = plsc.load_gather(feat_pairs[p], [s])
                        lo = plsc.bitcast(lax.shift_left(g32, 16), jnp.float32)
                        hi = plsc.bitcast(lax.bitwise_and(g32, himask), jnp.float32)
                        plsc.addupdate_scatter(agg_cols[2 * p], [t], lo)
                        plsc.addupdate_scatter(agg_cols[2 * p + 1], [t], hi)

                nxt = ck + _NBUF

                @pl.when(nxt < n_chunks)
                def _():
                    start(b, nxt)

        for half, ref in ((0, agg0_hbm), (1, agg1_hbm)):
            @pl.when(cid == half)
            def _():
                wb = [
                    pltpu.async_copy(
                        agg_cols[c],
                        ref.at[pl.ds((row0 + c) * n_nodes, n_nodes)], semf)
                    for c in range(cols_per_w)
                ]
                for d in wb:
                    d.wait()

    return k(featP_flat, edge_flat)


def _tc_linear_from_aggT(aggT0, aggT1, W, b2d, n_nodes, d_out):
    def body(a0_ref, a1_ref, w_ref, b_ref, out_ref):
        agg = a0_ref[...] + a1_ref[...]
        out_ref[...] = (
            lax.dot_general(
                agg, w_ref[...], (((0,), (1,)), ((), ())),
                preferred_element_type=jnp.float32,
            )
            + b_ref[...]
        )

    return pl.pallas_call(
        body,
        out_shape=jax.ShapeDtypeStruct((n_nodes, d_out), jnp.float32),
    )(aggT0, aggT1, W, b2d)


def kernel(feat, edge_index, W, b):
    n_nodes, d_in = feat.shape
    d_out = W.shape[0]
    fb = jax.lax.bitcast_convert_type(feat.astype(jnp.bfloat16), jnp.uint16)
    fb = fb.astype(jnp.uint32).T.reshape(d_in // 2, 2, n_nodes)
    featP_flat = (fb[:, 0] | (fb[:, 1] << 16)).astype(jnp.int32).reshape(-1)
    edge_flat = edge_index.reshape(-1)
    a0_flat, a1_flat = _sc_gather_scatter(featP_flat, edge_flat, n_nodes, d_in)
    aggT0 = a0_flat.reshape(d_in, n_nodes)
    aggT1 = a1_flat.reshape(d_in, n_nodes)
    return _tc_linear_from_aggT(aggT0, aggT1, W, b.reshape(1, d_out),
                                n_nodes, d_out)


# speedup vs baseline: 1.0800x; 1.0800x over previous
import functools

import jax
import jax.numpy as jnp
from jax import lax
from jax.experimental import pallas as pl
from jax.experimental.pallas import tpu as pltpu
from jax.experimental.pallas import tpu_sc as plsc

_NC = 2
_NS = 16
_L = 16
_NW = _NC * _NS

_CHUNK = 10000
_NBUF = 2


def _sc_gather_scatter(featT_flat, edge_flat, n_nodes, d_in):
    cols_per_w = d_in // _NW
    n_edges = edge_flat.shape[0] // 2
    n_chunks = n_edges // _CHUNK
    mesh = plsc.VectorSubcoreMesh(core_axis_name="c", subcore_axis_name="s")

    scratch = (
        [pltpu.VMEM((n_nodes,), jnp.float32) for _ in range(cols_per_w)]
        + [pltpu.VMEM((n_nodes,), jnp.float32) for _ in range(cols_per_w)]
        + [pltpu.VMEM((_CHUNK,), jnp.int32) for _ in range(2 * _NBUF)]
        + [pltpu.SemaphoreType.DMA, pltpu.SemaphoreType.DMA,
           pltpu.SemaphoreType.DMA]
    )

    @functools.partial(
        pl.kernel,
        out_type=jax.ShapeDtypeStruct((d_in * n_nodes,), jnp.float32),
        mesh=mesh,
        scratch_types=scratch,
        compiler_params=pltpu.CompilerParams(needs_layout_passes=False),
    )
    def k(featT_hbm, edge_hbm, aggT_hbm,
          f0, f1, f2, f3, a0, a1, a2, a3, s0, s1, t0, t1, sem0, sem1, semf):
        feat_cols = (f0, f1, f2, f3)
        agg_cols = (a0, a1, a2, a3)
        src_bufs = (s0, s1)
        dst_bufs = (t0, t1)
        sems = (sem0, sem1)
        wid = lax.axis_index("s") * _NC + lax.axis_index("c")
        row0 = wid * cols_per_w

        def start(b, ck):
            off = ck * _CHUNK
            pltpu.async_copy(edge_hbm.at[pl.ds(off, _CHUNK)], src_bufs[b], sems[b])
            pltpu.async_copy(edge_hbm.at[pl.ds(n_edges + off, _CHUNK)],
                             dst_bufs[b], sems[b])

        def drain(b):
            pltpu.make_async_copy(edge_hbm.at[pl.ds(0, _CHUNK)], src_bufs[b], sems[b]).wait()
            pltpu.make_async_copy(edge_hbm.at[pl.ds(0, _CHUNK)], dst_bufs[b], sems[b]).wait()

        for b in range(_NBUF):
            start(b, b)
        stage = [
            pltpu.async_copy(
                featT_hbm.at[pl.ds((row0 + c) * n_nodes, n_nodes)],
                feat_cols[c], semf)
            for c in range(cols_per_w)
        ]

        @plsc.parallel_loop(0, n_nodes // _L, unroll=8)
        def _zero(i):
            for c in range(cols_per_w):
                agg_cols[c][pl.ds(i * _L, _L)] = jnp.zeros((_L,), jnp.float32)

        for d in stage:
            d.wait()

        @pl.loop(0, n_chunks // _NBUF)
        def _outer(g):
            for b in range(_NBUF):
                ck = g * _NBUF + b
                drain(b)

                @plsc.parallel_loop(0, _CHUNK // _L, unroll=16)
                def _edges(i):
                    s = src_bufs[b][pl.ds(i * _L, _L)]
                    t = dst_bufs[b][pl.ds(i * _L, _L)]
                    for c in range(cols_per_w):
                        vals = plsc.load_gather(feat_cols[c], [s])
                        plsc.addupdate_scatter(agg_cols[c], [t], vals)

                nxt = ck + _NBUF

                @pl.when(nxt < n_chunks)
                def _():
                    start(b, nxt)

        wb = [
            pltpu.async_copy(
                agg_cols[c],
                aggT_hbm.at[pl.ds((row0 + c) * n_nodes, n_nodes)], semf)
            for c in range(cols_per_w)
        ]
        for d in wb:
            d.wait()

    return k(featT_flat, edge_flat)


def _tc_linear_from_aggT(aggT, W, b2d, n_nodes, d_out):
    def body(aggT_ref, w_ref, b_ref, out_ref):
        out_ref[...] = (
            lax.dot_general(
                aggT_ref[...], w_ref[...], (((0,), (1,)), ((), ())),
                preferred_element_type=jnp.float32,
            )
            + b_ref[...]
        )

    return pl.pallas_call(
        body,
        out_shape=jax.ShapeDtypeStruct((n_nodes, d_out), jnp.float32),
    )(aggT, W, b2d)


def kernel(feat, edge_index, W, b):
    n_nodes, d_in = feat.shape
    d_out = W.shape[0]
    featT_flat = feat.T.reshape(-1)
    edge_flat = edge_index.reshape(-1)
    aggT_flat = _sc_gather_scatter(featT_flat, edge_flat, n_nodes, d_in)
    aggT = aggT_flat.reshape(d_in, n_nodes)
    return _tc_linear_from_aggT(aggT, W, b.reshape(1, d_out), n_nodes, d_out)
